# baseline (speedup 1.0000x reference)
import functools

import jax
import jax.numpy as jnp
from jax import lax
from jax.experimental import pallas as pl
from jax.experimental.pallas import tpu as pltpu
from jax.experimental.pallas import tpu_sc as plsc

NUM_NODES = 1000000
EMBED_DIM = 64
BATCH = 16384
CHUNK = 31744


def _normalize_t_body(xt_ref, ot_ref, packed_ref):
    xt = xt_ref[...]
    ss = jnp.sum(xt * xt, axis=0, keepdims=True)
    n = jnp.maximum(jnp.sqrt(ss), 1e-12)
    yt = xt / n
    ot_ref[...] = yt
    yT = jnp.transpose(yt)
    packed_ref[...] = jnp.concatenate(
        [yT[: CHUNK // 2], yT[CHUNK // 2:]], axis=1)


@jax.jit
def _normalize_t(table_t):
    dim, n_rows = table_t.shape
    grid = pl.cdiv(n_rows, CHUNK)
    return pl.pallas_call(
        _normalize_t_body,
        grid=(grid,),
        in_specs=[pl.BlockSpec((dim, CHUNK), lambda i: (0, i))],
        out_specs=[
            pl.BlockSpec((dim, CHUNK), lambda i: (0, i)),
            pl.BlockSpec((CHUNK // 2, 128), lambda i: (i, 0)),
        ],
        out_shape=[
            jax.ShapeDtypeStruct((dim, n_rows), jnp.float32),
            jax.ShapeDtypeStruct((grid * (CHUNK // 2), 128), jnp.float32),
        ],
        compiler_params=pltpu.CompilerParams(
            vmem_limit_bytes=62 * 1024 * 1024),
    )(table_t)


@functools.lru_cache(maxsize=None)
def _make_gather2(n_rows, dim, batch):
    info = plsc.get_sparse_core_info()
    nc, ns = info.num_cores, info.num_subcores
    nw = nc * ns
    assert batch % (8 * nw) == 0
    b_per_w = batch // nw
    mesh = plsc.VectorSubcoreMesh(core_axis_name="c", subcore_axis_name="s")

    @functools.partial(
        pl.kernel,
        mesh=mesh,
        out_type=[
            jax.ShapeDtypeStruct((batch // 2, 2 * dim), jnp.float32),
            jax.ShapeDtypeStruct((batch // 2, 2 * dim), jnp.float32),
        ],
        scratch_types=[
            pltpu.VMEM((b_per_w,), jnp.int32),
            pltpu.VMEM((b_per_w, dim), jnp.float32),
            pltpu.VMEM((b_per_w,), jnp.int32),
            pltpu.VMEM((b_per_w, dim), jnp.float32),
            pltpu.SemaphoreType.DMA,
            pltpu.SemaphoreType.DMA,
        ],
        compiler_params=pltpu.CompilerParams(use_tc_tiling_on_sc=False),
    )
    def gather2(table_hbm, idx_hbm, xout_hbm, yout_hbm,
                xidx_v, xrows_v, yidx_v, yrows_v, xsem, ysem):
        wid = lax.axis_index("s") * nc + lax.axis_index("c")
        base = wid * b_per_w
        pltpu.sync_copy(idx_hbm.at[pl.ds(base, b_per_w)], xidx_v)
        pltpu.sync_copy(idx_hbm.at[pl.ds(batch + base, b_per_w)], yidx_v)
        xcp = pltpu.async_copy(table_hbm.at[xidx_v], xrows_v, xsem)
        ycp = pltpu.async_copy(table_hbm.at[yidx_v], yrows_v, ysem)
        half = wid // (nw // 2)
        prow = (wid % (nw // 2)) * b_per_w
        xcp.wait()
        ycp.wait()

        @pl.when(half == 0)
        def _():
            pltpu.sync_copy(xrows_v, xout_hbm.at[pl.ds(prow, b_per_w),
                                                 pl.ds(0, dim)])
            pltpu.sync_copy(yrows_v, yout_hbm.at[pl.ds(prow, b_per_w),
                                                 pl.ds(0, dim)])

        @pl.when(half == 1)
        def _():
            pltpu.sync_copy(xrows_v, xout_hbm.at[pl.ds(prow, b_per_w),
                                                 pl.ds(dim, dim)])
            pltpu.sync_copy(yrows_v, yout_hbm.at[pl.ds(prow, b_per_w),
                                                 pl.ds(dim, dim)])

    return gather2


def _finish_body(xw_ref, yw_ref, xo_ref, yo_ref):
    half = BATCH // 2
    xwt = jnp.transpose(xw_ref[...])
    xo_ref[:, 0:half] = xwt[0:EMBED_DIM]
    xo_ref[:, half:BATCH] = xwt[EMBED_DIM:2 * EMBED_DIM]
    ywt = jnp.transpose(yw_ref[...])
    yo_ref[:, 0:half] = ywt[0:EMBED_DIM]
    yo_ref[:, half:BATCH] = ywt[EMBED_DIM:2 * EMBED_DIM]


@jax.jit
def _finish(xw, yw):
    half, width = xw.shape
    return pl.pallas_call(
        _finish_body,
        in_specs=[
            pl.BlockSpec((half, width), lambda: (0, 0)),
            pl.BlockSpec((half, width), lambda: (0, 0)),
        ],
        out_specs=[
            pl.BlockSpec((EMBED_DIM, BATCH), lambda: (0, 0)),
            pl.BlockSpec((EMBED_DIM, BATCH), lambda: (0, 0)),
        ],
        out_shape=[
            jax.ShapeDtypeStruct((EMBED_DIM, BATCH), jnp.float32),
            jax.ShapeDtypeStruct((EMBED_DIM, BATCH), jnp.float32),
        ],
    )(xw, yw)


def _remap(idx):
    half = CHUNK // 2
    off = idx % CHUNK
    q = (idx // CHUNK) * half + off % half
    return 2 * q + off // half


def kernel(x_inputs, y_inputs, embedding):
    idx = _remap(jnp.concatenate(
        [x_inputs.astype(jnp.int32), y_inputs.astype(jnp.int32)]))
    norm_t, packed = _normalize_t(jnp.swapaxes(embedding, 0, 1))
    last_embed = jnp.swapaxes(norm_t, 0, 1)
    table = packed.reshape(packed.shape[0] * 2, EMBED_DIM)
    xw, yw = _make_gather2(table.shape[0], EMBED_DIM, BATCH)(table, idx)
    xo, yo = _finish(xw, yw)
    last_x = jnp.swapaxes(xo, 0, 1)
    last_y = jnp.swapaxes(yo, 0, 1)
    return (last_x, last_y, last_embed)


# speedup vs baseline: 1.0019x; 1.0019x over previous
import functools

import jax
import jax.numpy as jnp
from jax import lax
from jax.experimental import pallas as pl
from jax.experimental.pallas import tpu as pltpu
from jax.experimental.pallas import tpu_sc as plsc

NUM_NODES = 1000000
EMBED_DIM = 64
BATCH = 16384
CHUNK = 29696


def _normalize_t_body(xt_ref, ot_ref, packed_ref):
    xt = xt_ref[...]
    ss = jnp.sum(xt * xt, axis=0, keepdims=True)
    n = jnp.maximum(jnp.sqrt(ss), 1e-12)
    yt = xt / n
    ot_ref[...] = yt
    yT = jnp.transpose(yt)
    packed_ref[...] = jnp.concatenate(
        [yT[: CHUNK // 2], yT[CHUNK // 2:]], axis=1)


@jax.jit
def _normalize_t(table_t):
    dim, n_rows = table_t.shape
    grid = pl.cdiv(n_rows, CHUNK)
    return pl.pallas_call(
        _normalize_t_body,
        grid=(grid,),
        in_specs=[pl.BlockSpec((dim, CHUNK), lambda i: (0, i))],
        out_specs=[
            pl.BlockSpec((dim, CHUNK), lambda i: (0, i)),
            pl.BlockSpec((CHUNK // 2, 128), lambda i: (i, 0)),
        ],
        out_shape=[
            jax.ShapeDtypeStruct((dim, n_rows), jnp.float32),
            jax.ShapeDtypeStruct((grid * (CHUNK // 2), 128), jnp.float32),
        ],
    )(table_t)


@functools.lru_cache(maxsize=None)
def _make_gather2(n_rows, dim, batch):
    info = plsc.get_sparse_core_info()
    nc, ns = info.num_cores, info.num_subcores
    nw = nc * ns
    assert batch % (8 * nw) == 0
    b_per_w = batch // nw
    mesh = plsc.VectorSubcoreMesh(core_axis_name="c", subcore_axis_name="s")

    @functools.partial(
        pl.kernel,
        mesh=mesh,
        out_type=[
            jax.ShapeDtypeStruct((batch // 2, 2 * dim), jnp.float32),
            jax.ShapeDtypeStruct((batch // 2, 2 * dim), jnp.float32),
        ],
        scratch_types=[
            pltpu.VMEM((b_per_w,), jnp.int32),
            pltpu.VMEM((b_per_w, dim), jnp.float32),
            pltpu.VMEM((b_per_w,), jnp.int32),
            pltpu.VMEM((b_per_w, dim), jnp.float32),
            pltpu.SemaphoreType.DMA,
            pltpu.SemaphoreType.DMA,
        ],
        compiler_params=pltpu.CompilerParams(use_tc_tiling_on_sc=False),
    )
    def gather2(table_hbm, idx_hbm, xout_hbm, yout_hbm,
                xidx_v, xrows_v, yidx_v, yrows_v, xsem, ysem):
        wid = lax.axis_index("s") * nc + lax.axis_index("c")
        base = wid * b_per_w
        pltpu.sync_copy(idx_hbm.at[pl.ds(base, b_per_w)], xidx_v)
        pltpu.sync_copy(idx_hbm.at[pl.ds(batch + base, b_per_w)], yidx_v)
        xcp = pltpu.async_copy(table_hbm.at[xidx_v], xrows_v, xsem)
        ycp = pltpu.async_copy(table_hbm.at[yidx_v], yrows_v, ysem)
        half = wid // (nw // 2)
        prow = (wid % (nw // 2)) * b_per_w
        xcp.wait()
        ycp.wait()

        @pl.when(half == 0)
        def _():
            pltpu.sync_copy(xrows_v, xout_hbm.at[pl.ds(prow, b_per_w),
                                                 pl.ds(0, dim)])
            pltpu.sync_copy(yrows_v, yout_hbm.at[pl.ds(prow, b_per_w),
                                                 pl.ds(0, dim)])

        @pl.when(half == 1)
        def _():
            pltpu.sync_copy(xrows_v, xout_hbm.at[pl.ds(prow, b_per_w),
                                                 pl.ds(dim, dim)])
            pltpu.sync_copy(yrows_v, yout_hbm.at[pl.ds(prow, b_per_w),
                                                 pl.ds(dim, dim)])

    return gather2


def _finish_body(xw_ref, yw_ref, xo_ref, yo_ref):
    half = BATCH // 2
    xwt = jnp.transpose(xw_ref[...])
    xo_ref[:, 0:half] = xwt[0:EMBED_DIM]
    xo_ref[:, half:BATCH] = xwt[EMBED_DIM:2 * EMBED_DIM]
    ywt = jnp.transpose(yw_ref[...])
    yo_ref[:, 0:half] = ywt[0:EMBED_DIM]
    yo_ref[:, half:BATCH] = ywt[EMBED_DIM:2 * EMBED_DIM]


@jax.jit
def _finish(xw, yw):
    half, width = xw.shape
    return pl.pallas_call(
        _finish_body,
        in_specs=[
            pl.BlockSpec((half, width), lambda: (0, 0)),
            pl.BlockSpec((half, width), lambda: (0, 0)),
        ],
        out_specs=[
            pl.BlockSpec((EMBED_DIM, BATCH), lambda: (0, 0)),
            pl.BlockSpec((EMBED_DIM, BATCH), lambda: (0, 0)),
        ],
        out_shape=[
            jax.ShapeDtypeStruct((EMBED_DIM, BATCH), jnp.float32),
            jax.ShapeDtypeStruct((EMBED_DIM, BATCH), jnp.float32),
        ],
    )(xw, yw)


def _remap(idx):
    half = CHUNK // 2
    off = idx % CHUNK
    q = (idx // CHUNK) * half + off % half
    return 2 * q + off // half


def kernel(x_inputs, y_inputs, embedding):
    idx = _remap(jnp.concatenate(
        [x_inputs.astype(jnp.int32), y_inputs.astype(jnp.int32)]))
    norm_t, packed = _normalize_t(jnp.swapaxes(embedding, 0, 1))
    last_embed = jnp.swapaxes(norm_t, 0, 1)
    table = packed.reshape(packed.shape[0] * 2, EMBED_DIM)
    xw, yw = _make_gather2(table.shape[0], EMBED_DIM, BATCH)(table, idx)
    xo, yo = _finish(xw, yw)
    last_x = jnp.swapaxes(xo, 0, 1)
    last_y = jnp.swapaxes(yo, 0, 1)
    return (last_x, last_y, last_embed)
